# baseline (speedup 1.0000x reference)
import jax
import jax.numpy as jnp
from jax.experimental import pallas as pl

N_GRAPHS = 64
NUM_BLOCKS = 2


def _head_matmul_kernel(h_ref, w_ref, b_ref, o_ref):
    o_ref[...] = h_ref[...] @ w_ref[...] + b_ref[...]


def kernel(x, edge_index, edge_attr, batch_idx, emb_W, emb_b, rgcn_Wrel, rgcn_Wroot, rgcn_b, mf_Wl, mf_bl, mf_Wr, head_W1, head_b1, head_W2, head_b2):
    N = x.shape[0]
    R = rgcn_Wrel.shape[0]
    src = edge_index[0]
    dst = edge_index[1]
    rel = jnp.argmax(edge_attr, axis=-1)
    node_ids = jnp.arange(N)
    deg = jnp.clip(jnp.bincount(dst, length=N), 0, mf_Wl.shape[0] - 1)
    key_ = dst * R + rel
    cnt = jnp.bincount(key_, length=N * R).astype(jnp.float32)
    norm = 1.0 / jnp.maximum(cnt[key_], 1.0)

    out = x @ emb_W + emb_b
    for _ in range(NUM_BLOCKS):
        out = jax.nn.relu(out)
        h_all = jnp.einsum('nd,rdf->rnf', out, rgcn_Wrel)
        msg = h_all[rel, src]
        agg = jax.ops.segment_sum(msg * norm[:, None], dst, num_segments=N)
        out = agg + out @ rgcn_Wroot + rgcn_b
        out = jax.nn.relu(out)
        nbr = jax.ops.segment_sum(out[src], dst, num_segments=N)
        hl = jnp.einsum('nd,kdf->knf', out, mf_Wl)
        hr = jnp.einsum('nd,kdf->knf', nbr, mf_Wr)
        out = hl[deg, node_ids] + mf_bl[deg] + hr[deg, node_ids]

    pooled = jax.ops.segment_sum(out, batch_idx, num_segments=N_GRAPHS)
    h = jax.nn.relu(pooled @ head_W1 + head_b1)
    return pl.pallas_call(
        _head_matmul_kernel,
        out_shape=jax.ShapeDtypeStruct((N_GRAPHS, head_W2.shape[1]), jnp.float32),
    )(h, head_W2, head_b2)


# speedup vs baseline: 2.4898x; 2.4898x over previous
import dataclasses
import functools

import jax
import jax.numpy as jnp
from jax import lax
from jax.experimental import pallas as pl
from jax.experimental.pallas import tpu as pltpu
from jax.experimental.pallas import tpu_sc as plsc

N = 10000
E = 320000
F = 128
R = 16
NG = 64
NDEG = 11

NC = 2
NS = 16
NW = NC * NS
EW = E // NW
CK = 80
NCH = EW // CK
NPAD = 10240

_MESH = plsc.VectorSubcoreMesh(core_axis_name="c", subcore_axis_name="s")

_SC_CP = pltpu.CompilerParams()
if "needs_layout_passes" in pltpu.CompilerParams.__dataclass_fields__:
    _SC_CP = dataclasses.replace(_SC_CP, needs_layout_passes=False)


def _k0_body(a_ref, s_ref, d_ref, g_ref, k_ref):
    best = a_ref[0:1, :]
    bidx = jnp.zeros(best.shape, jnp.int32)
    for k in range(1, R):
        v = a_ref[k:k + 1, :]
        gt = v > best
        bidx = jnp.where(gt, k, bidx)
        best = jnp.where(gt, v, best)
    src = s_ref[0]
    dst = d_ref[0]
    g_ref[0] = bidx * N + src
    k_ref[0] = dst * R + bidx


def _k0(attr_t, src3, dst3):
    w = 2560
    nblk = attr_t.shape[1] // w
    return pl.pallas_call(
        _k0_body,
        grid=(nblk,),
        in_specs=[
            pl.BlockSpec((R, w), lambda i: (0, i)),
            pl.BlockSpec((1, 1, w), lambda i: (i, 0, 0)),
            pl.BlockSpec((1, 1, w), lambda i: (i, 0, 0)),
        ],
        out_specs=[
            pl.BlockSpec((1, 1, w), lambda i: (i, 0, 0)),
            pl.BlockSpec((1, 1, w), lambda i: (i, 0, 0)),
        ],
        out_shape=[
            jax.ShapeDtypeStruct((nblk, 1, w), jnp.int32),
            jax.ShapeDtypeStruct((nblk, 1, w), jnp.int32),
        ],
    )(attr_t, src3, dst3)


@functools.partial(
    pl.kernel,
    mesh=_MESH,
    out_type=jax.ShapeDtypeStruct((NC * N * R,), jnp.float32),
    compiler_params=_SC_CP,
    scratch_types=[
        pltpu.VMEM((2000,), jnp.int32),
        pltpu.VMEM((10000,), jnp.float32),
    ],
)
def _s1(key_hbm, cnt_out, kbuf, cnt_v):
    c = lax.axis_index("c")
    s = lax.axis_index("s")
    lo = s * 10000
    ones16 = jnp.full((16,), 1.0, jnp.float32)
    zeros16 = jnp.zeros((16,), jnp.float32)

    @pl.loop(0, 625)
    def _(i):
        cnt_v[pl.ds(i * 16, 16)] = zeros16

    ehalf = E // NC

    @pl.loop(0, ehalf // 2000)
    def _(i):
        pltpu.sync_copy(key_hbm.at[pl.ds(c * ehalf + i * 2000, 2000)], kbuf)

        @pl.loop(0, 125)
        def _(m):
            kv16 = kbuf[pl.ds(m * 16, 16)]
            rel = kv16 - lo
            mask = (rel >= 0) & (rel < 10000)
            relc = jnp.where(mask, rel, 0)
            plsc.addupdate_scatter(cnt_v, [relc], ones16, mask=mask)

    pltpu.sync_copy(cnt_v, cnt_out.at[pl.ds(c * (N * R) + lo, 10000)])


@functools.partial(
    pl.kernel,
    mesh=_MESH,
    out_type=jax.ShapeDtypeStruct((NC * NPAD, F), jnp.float32),
    compiler_params=_SC_CP,
    scratch_types=[
        pltpu.VMEM((CK,), jnp.int32),
        pltpu.VMEM((CK,), jnp.int32),
        pltpu.VMEM((CK,), jnp.int32),
        pltpu.VMEM((CK,), jnp.float32),
        pltpu.VMEM((CK, F), jnp.float32),
        pltpu.VMEM_SHARED((NPAD, F), jnp.float32),
    ],
)
def _s3(gidx_hbm, key_hbm, dst_hbm, inv_hbm, hall_hbm, z2_hbm, agg_out,
        gv, kv, dv, nv, rows, agg_sp):
    c = lax.axis_index("c")
    s = lax.axis_index("s")
    w = s * NC + c

    pltpu.sync_copy(z2_hbm, agg_sp.at[pl.ds(s * 640, 640)])
    plsc.subcore_barrier()
    lanes = lax.iota(jnp.int32, 16)

    @pl.loop(0, NCH)
    def _(ch):
        base = w * EW + ch * CK
        pltpu.sync_copy(gidx_hbm.at[pl.ds(base, CK)], gv)
        pltpu.sync_copy(key_hbm.at[pl.ds(base, CK)], kv)
        pltpu.sync_copy(dst_hbm.at[pl.ds(base, CK)], dv)
        pltpu.sync_copy(inv_hbm.at[kv], nv)
        pltpu.sync_copy(hall_hbm.at[gv], rows)

        @pl.loop(0, CK)
        def _(j):
            j16 = jnp.full((16,), j, jnp.int32)
            bc = plsc.load_gather(nv, [j16])
            for t in range(F // 16):
                cols = lanes + (t * 16)
                v = plsc.load_gather(rows, [j16, cols])
                plsc.store_scatter(rows, [j16, cols], v * bc)

        pltpu.sync_copy(rows, agg_sp.at[dv], add=True)

    plsc.subcore_barrier()

    pltpu.sync_copy(agg_sp.at[pl.ds(s * 640, 640)],
                    agg_out.at[pl.ds(c * NPAD + s * 640, 640)])


@functools.partial(
    pl.kernel,
    mesh=_MESH,
    out_type=jax.ShapeDtypeStruct((NC * NPAD, F), jnp.float32),
    scratch_types=[
        pltpu.VMEM((CK,), jnp.int32),
        pltpu.VMEM((CK,), jnp.int32),
        pltpu.VMEM((CK, F), jnp.float32),
        pltpu.VMEM_SHARED((NPAD, F), jnp.float32),
    ],
)
def _s4(src_hbm, dst_hbm, xm_hbm, z2_hbm, nbr_out, sv, dv, rows, nbr_sp):
    c = lax.axis_index("c")
    s = lax.axis_index("s")
    w = s * NC + c

    pltpu.sync_copy(z2_hbm, nbr_sp.at[pl.ds(s * 640, 640)])
    plsc.subcore_barrier()

    @pl.loop(0, NCH)
    def _(ch):
        base = w * EW + ch * CK
        pltpu.sync_copy(src_hbm.at[pl.ds(base, CK)], sv)
        pltpu.sync_copy(dst_hbm.at[pl.ds(base, CK)], dv)
        pltpu.sync_copy(xm_hbm.at[sv], rows)
        pltpu.sync_copy(rows, nbr_sp.at[dv], add=True)

    plsc.subcore_barrier()

    pltpu.sync_copy(nbr_sp.at[pl.ds(s * 640, 640)],
                    nbr_out.at[pl.ds(c * NPAD + s * 640, 640)])


def _k1_body(x_ref, w_ref, b_ref, o_ref):
    o_ref[...] = jnp.dot(x_ref[...], w_ref[...],
                         preferred_element_type=jnp.float32) + b_ref[...]


def _k1(x, w, b2):
    return pl.pallas_call(
        _k1_body,
        grid=(10,),
        in_specs=[
            pl.BlockSpec((1000, F), lambda i: (i, 0)),
            pl.BlockSpec((F, F), lambda i: (0, 0)),
            pl.BlockSpec((1, F), lambda i: (0, 0)),
        ],
        out_specs=pl.BlockSpec((1000, F), lambda i: (i, 0)),
        out_shape=jax.ShapeDtypeStruct((N, F), jnp.float32),
    )(x, w, b2)


def _k1b_body(c_ref, o_ref):
    cs = c_ref[0] + c_ref[1]
    o_ref[...] = 1.0 / jnp.maximum(cs, 1.0)


def _k1b(cnt2):
    return pl.pallas_call(
        _k1b_body,
        out_shape=jax.ShapeDtypeStruct((1250, 128), jnp.float32),
    )(cnt2)


def _k1c_body(c_ref, o_ref):
    d = jnp.sum(c_ref[0] + c_ref[1], axis=1, keepdims=True)
    o_ref[...] = jnp.clip(d, 0.0, float(NDEG - 1))


def _k1c(cnt2d):
    return pl.pallas_call(
        _k1c_body,
        grid=(10,),
        in_specs=[pl.BlockSpec((2, 1000, 16), lambda i: (0, i, 0))],
        out_specs=pl.BlockSpec((1000, 1), lambda i: (i, 0)),
        out_shape=jax.ShapeDtypeStruct((N, 1), jnp.float32),
    )(cnt2d)


def _k2_body(x_ref, w_ref, o_ref):
    xr = jnp.maximum(x_ref[...], 0.0)
    for r in range(R):
        o_ref[r] = jnp.dot(xr, w_ref[r], preferred_element_type=jnp.float32)


def _k2(prev, wrel):
    return pl.pallas_call(
        _k2_body,
        grid=(10,),
        in_specs=[
            pl.BlockSpec((1000, F), lambda i: (i, 0)),
            pl.BlockSpec((R, F, F), lambda i: (0, 0, 0)),
        ],
        out_specs=pl.BlockSpec((R, 1000, F), lambda i: (0, i, 0)),
        out_shape=jax.ShapeDtypeStruct((R, N, F), jnp.float32),
    )(prev, wrel)


def _k3_body(x_ref, a_ref, w_ref, b_ref, o_ref):
    xr = jnp.maximum(x_ref[...], 0.0)
    root = jnp.dot(xr, w_ref[...], preferred_element_type=jnp.float32)
    o_ref[...] = jnp.maximum(a_ref[0] + a_ref[1] + root + b_ref[...], 0.0)


def _k3(prev, agg2, wroot, b2):
    return pl.pallas_call(
        _k3_body,
        grid=(10,),
        in_specs=[
            pl.BlockSpec((1000, F), lambda i: (i, 0)),
            pl.BlockSpec((2, 1000, F), lambda i: (0, i, 0)),
            pl.BlockSpec((F, F), lambda i: (0, 0)),
            pl.BlockSpec((1, F), lambda i: (0, 0)),
        ],
        out_specs=pl.BlockSpec((1000, F), lambda i: (i, 0)),
        out_shape=jax.ShapeDtypeStruct((N, F), jnp.float32),
    )(prev, agg2, wroot, b2)


def _k4_body(x_ref, n_ref, d_ref, wl_ref, bl_ref, wr_ref, o_ref):
    x = x_ref[...]
    nb = n_ref[0] + n_ref[1]
    d = d_ref[...]
    acc = jnp.zeros(o_ref.shape, jnp.float32)
    for k in range(NDEG):
        hk = (jnp.dot(x, wl_ref[k], preferred_element_type=jnp.float32)
              + jnp.dot(nb, wr_ref[k], preferred_element_type=jnp.float32)
              + bl_ref[k][None, :])
        acc = acc + jnp.where(d == float(k), hk, 0.0)
    o_ref[...] = acc


def _k4(xm, nbr2, degc, wl, bl, wr):
    return pl.pallas_call(
        _k4_body,
        grid=(10,),
        in_specs=[
            pl.BlockSpec((1000, F), lambda i: (i, 0)),
            pl.BlockSpec((2, 1000, F), lambda i: (0, i, 0)),
            pl.BlockSpec((1000, 1), lambda i: (i, 0)),
            pl.BlockSpec((NDEG, F, F), lambda i: (0, 0, 0)),
            pl.BlockSpec((NDEG, F), lambda i: (0, 0)),
            pl.BlockSpec((NDEG, F, F), lambda i: (0, 0, 0)),
        ],
        out_specs=pl.BlockSpec((1000, F), lambda i: (i, 0)),
        out_shape=jax.ShapeDtypeStruct((N, F), jnp.float32),
    )(xm, nbr2, degc, wl, bl, wr)


def _k5_body(x_ref, b_ref, o_ref):
    @pl.when(pl.program_id(0) == 0)
    def _():
        o_ref[...] = jnp.zeros(o_ref.shape, jnp.float32)

    bi = b_ref[0]
    gids = lax.broadcasted_iota(jnp.int32, (NG, 1000), 0)
    oh = (jnp.broadcast_to(bi, (NG, 1000)) == gids).astype(jnp.float32)
    o_ref[...] += jnp.dot(oh, x_ref[...], preferred_element_type=jnp.float32)


def _k5(out2, batch3):
    return pl.pallas_call(
        _k5_body,
        grid=(10,),
        in_specs=[
            pl.BlockSpec((1000, F), lambda i: (i, 0)),
            pl.BlockSpec((1, 1, 1000), lambda i: (i, 0, 0)),
        ],
        out_specs=pl.BlockSpec((NG, F), lambda i: (0, 0)),
        out_shape=jax.ShapeDtypeStruct((NG, F), jnp.float32),
    )(out2, batch3)


def _k6_body(p_ref, w1_ref, b1_ref, w2_ref, b2_ref, o_ref):
    h = jnp.maximum(jnp.dot(p_ref[...], w1_ref[...],
                            preferred_element_type=jnp.float32) + b1_ref[...],
                    0.0)
    o_ref[...] = jnp.dot(h, w2_ref[...],
                         preferred_element_type=jnp.float32) + b2_ref[...]


def _k6(pooled, w1, b1_2, w2, b2_2):
    return pl.pallas_call(
        _k6_body,
        out_shape=jax.ShapeDtypeStruct((NG, w2.shape[1]), jnp.float32),
    )(pooled, w1, b1_2, w2, b2_2)


def kernel(x, edge_index, edge_attr, batch_idx, emb_W, emb_b, rgcn_Wrel,
           rgcn_Wroot, rgcn_b, mf_Wl, mf_bl, mf_Wr, head_W1, head_b1,
           head_W2, head_b2):
    src_f = edge_index[0]
    dst_f = edge_index[1]
    attr_t = edge_attr.T
    src3 = src_f.reshape(125, 1, 2560)
    dst3 = dst_f.reshape(125, 1, 2560)

    gidx3, key3 = _k0(attr_t, src3, dst3)
    gidx_f = gidx3.reshape(E)
    key_f = key3.reshape(E)

    z2 = jnp.zeros((640, F), jnp.float32)

    cnt_p = _s1(key_f)
    inv = _k1b(cnt_p.reshape(2, 1250, 128)).reshape(N * R)
    degc = _k1c(cnt_p.reshape(2, N, R))

    out = _k1(x, emb_W, emb_b.reshape(1, F))

    for _ in range(2):
        h_all = _k2(out, rgcn_Wrel).reshape(R * N, F)
        agg2 = _s3(gidx_f, key_f, dst_f, inv, h_all, z2).reshape(2, NPAD, F)
        xm = _k3(out, agg2, rgcn_Wroot, rgcn_b.reshape(1, F))
        nbr2 = _s4(src_f, dst_f, xm, z2).reshape(2, NPAD, F)
        out = _k4(xm, nbr2, degc, mf_Wl, mf_bl, mf_Wr)

    pooled = _k5(out, batch_idx.reshape(10, 1, 1000))
    return _k6(pooled, head_W1, head_b1.reshape(1, F),
               head_W2, head_b2.reshape(1, head_W2.shape[1]))


# speedup vs baseline: 3.2136x; 1.2907x over previous
import dataclasses
import functools

import jax
import jax.numpy as jnp
from jax import lax
from jax.experimental import pallas as pl
from jax.experimental.pallas import tpu as pltpu
from jax.experimental.pallas import tpu_sc as plsc

N = 10000
E = 320000
F = 128
R = 16
NG = 64
NDEG = 11

NC = 2
NS = 16
NW = NC * NS
EW = E // NW
CK = 80
NCH = EW // CK
NPAD = 10240

_MESH = plsc.VectorSubcoreMesh(core_axis_name="c", subcore_axis_name="s")

_SC_CP = pltpu.CompilerParams()
if "needs_layout_passes" in pltpu.CompilerParams.__dataclass_fields__:
    _SC_CP = dataclasses.replace(_SC_CP, needs_layout_passes=False)


def _k0_body(a_ref, s_ref, d_ref, g_ref, k_ref):
    best = a_ref[0:1, :]
    bidx = jnp.zeros(best.shape, jnp.int32)
    for k in range(1, R):
        v = a_ref[k:k + 1, :]
        gt = v > best
        bidx = jnp.where(gt, k, bidx)
        best = jnp.where(gt, v, best)
    src = s_ref[0]
    dst = d_ref[0]
    g_ref[0] = bidx * N + src
    k_ref[0] = dst * R + bidx


def _k0(attr_t, src3, dst3):
    w = 2560
    nblk = attr_t.shape[1] // w
    return pl.pallas_call(
        _k0_body,
        grid=(nblk,),
        in_specs=[
            pl.BlockSpec((R, w), lambda i: (0, i)),
            pl.BlockSpec((1, 1, w), lambda i: (i, 0, 0)),
            pl.BlockSpec((1, 1, w), lambda i: (i, 0, 0)),
        ],
        out_specs=[
            pl.BlockSpec((1, 1, w), lambda i: (i, 0, 0)),
            pl.BlockSpec((1, 1, w), lambda i: (i, 0, 0)),
        ],
        out_shape=[
            jax.ShapeDtypeStruct((nblk, 1, w), jnp.int32),
            jax.ShapeDtypeStruct((nblk, 1, w), jnp.int32),
        ],
    )(attr_t, src3, dst3)


@functools.partial(
    pl.kernel,
    mesh=_MESH,
    out_type=jax.ShapeDtypeStruct((NC * N * R,), jnp.float32),
    compiler_params=_SC_CP,
    scratch_types=[
        pltpu.VMEM((2000,), jnp.int32),
        pltpu.VMEM((10000,), jnp.float32),
    ],
)
def _s1(key_hbm, cnt_out, kbuf, cnt_v):
    c = lax.axis_index("c")
    s = lax.axis_index("s")
    lo = s * 10000
    ones16 = jnp.full((16,), 1.0, jnp.float32)
    zeros16 = jnp.zeros((16,), jnp.float32)

    @pl.loop(0, 625)
    def _(i):
        cnt_v[pl.ds(i * 16, 16)] = zeros16

    ehalf = E // NC

    @pl.loop(0, ehalf // 2000)
    def _(i):
        pltpu.sync_copy(key_hbm.at[pl.ds(c * ehalf + i * 2000, 2000)], kbuf)

        @pl.loop(0, 125)
        def _(m):
            kv16 = kbuf[pl.ds(m * 16, 16)]
            rel = kv16 - lo
            mask = (rel >= 0) & (rel < 10000)
            relc = jnp.where(mask, rel, 0)
            plsc.addupdate_scatter(cnt_v, [relc], ones16, mask=mask)

    pltpu.sync_copy(cnt_v, cnt_out.at[pl.ds(c * (N * R) + lo, 10000)])


@functools.partial(
    pl.kernel,
    mesh=_MESH,
    out_type=jax.ShapeDtypeStruct((NC * NPAD, F), jnp.float32),
    compiler_params=_SC_CP,
    scratch_types=[
        pltpu.VMEM((CK,), jnp.int32),
        pltpu.VMEM((CK,), jnp.int32),
        pltpu.VMEM((CK,), jnp.int32),
        pltpu.VMEM((CK,), jnp.float32),
        pltpu.VMEM((CK, F), jnp.float32),
        pltpu.VMEM_SHARED((NPAD, F), jnp.float32),
    ],
)
def _s3(gidx_hbm, key_hbm, dst_hbm, inv_hbm, hall_hbm, z2_hbm, agg_out,
        gv, kv, dv, nv, rows, agg_sp):
    c = lax.axis_index("c")
    s = lax.axis_index("s")
    w = s * NC + c

    pltpu.sync_copy(z2_hbm, agg_sp.at[pl.ds(s * 640, 640)])
    plsc.subcore_barrier()
    lanes = lax.iota(jnp.int32, 16)

    @pl.loop(0, NCH)
    def _(ch):
        base = w * EW + ch * CK
        pltpu.sync_copy(gidx_hbm.at[pl.ds(base, CK)], gv)
        pltpu.sync_copy(key_hbm.at[pl.ds(base, CK)], kv)
        pltpu.sync_copy(dst_hbm.at[pl.ds(base, CK)], dv)
        pltpu.sync_copy(inv_hbm.at[kv], nv)
        pltpu.sync_copy(hall_hbm.at[gv], rows)

        @pl.loop(0, CK)
        def _(j):
            j16 = jnp.full((16,), j, jnp.int32)
            bc = plsc.load_gather(nv, [j16])
            row = rows.at[j]
            for t in range(F // 16):
                sl = pl.ds(t * 16, 16)
                row[sl] = row[sl] * bc

        pltpu.sync_copy(rows, agg_sp.at[dv], add=True)

    plsc.subcore_barrier()

    pltpu.sync_copy(agg_sp.at[pl.ds(s * 640, 640)],
                    agg_out.at[pl.ds(c * NPAD + s * 640, 640)])


@functools.partial(
    pl.kernel,
    mesh=_MESH,
    out_type=jax.ShapeDtypeStruct((NC * NPAD, F), jnp.float32),
    scratch_types=[
        pltpu.VMEM((CK,), jnp.int32),
        pltpu.VMEM((CK,), jnp.int32),
        pltpu.VMEM((CK, F), jnp.float32),
        pltpu.VMEM_SHARED((NPAD, F), jnp.float32),
    ],
)
def _s4(src_hbm, dst_hbm, xm_hbm, z2_hbm, nbr_out, sv, dv, rows, nbr_sp):
    c = lax.axis_index("c")
    s = lax.axis_index("s")
    w = s * NC + c

    pltpu.sync_copy(z2_hbm, nbr_sp.at[pl.ds(s * 640, 640)])
    plsc.subcore_barrier()

    @pl.loop(0, NCH)
    def _(ch):
        base = w * EW + ch * CK
        pltpu.sync_copy(src_hbm.at[pl.ds(base, CK)], sv)
        pltpu.sync_copy(dst_hbm.at[pl.ds(base, CK)], dv)
        pltpu.sync_copy(xm_hbm.at[sv], rows)
        pltpu.sync_copy(rows, nbr_sp.at[dv], add=True)

    plsc.subcore_barrier()

    pltpu.sync_copy(nbr_sp.at[pl.ds(s * 640, 640)],
                    nbr_out.at[pl.ds(c * NPAD + s * 640, 640)])


def _k1_body(x_ref, w_ref, b_ref, o_ref):
    o_ref[...] = jnp.dot(x_ref[...], w_ref[...],
                         preferred_element_type=jnp.float32) + b_ref[...]


def _k1(x, w, b2):
    return pl.pallas_call(
        _k1_body,
        grid=(10,),
        in_specs=[
            pl.BlockSpec((1000, F), lambda i: (i, 0)),
            pl.BlockSpec((F, F), lambda i: (0, 0)),
            pl.BlockSpec((1, F), lambda i: (0, 0)),
        ],
        out_specs=pl.BlockSpec((1000, F), lambda i: (i, 0)),
        out_shape=jax.ShapeDtypeStruct((N, F), jnp.float32),
    )(x, w, b2)


def _k1b_body(c_ref, o_ref):
    cs = c_ref[0] + c_ref[1]
    o_ref[...] = 1.0 / jnp.maximum(cs, 1.0)


def _k1b(cnt2):
    return pl.pallas_call(
        _k1b_body,
        out_shape=jax.ShapeDtypeStruct((1250, 128), jnp.float32),
    )(cnt2)


def _k1c_body(c_ref, o_ref):
    d = jnp.sum(c_ref[0] + c_ref[1], axis=1, keepdims=True)
    o_ref[...] = jnp.clip(d, 0.0, float(NDEG - 1))


def _k1c(cnt2d):
    return pl.pallas_call(
        _k1c_body,
        grid=(10,),
        in_specs=[pl.BlockSpec((2, 1000, 16), lambda i: (0, i, 0))],
        out_specs=pl.BlockSpec((1000, 1), lambda i: (i, 0)),
        out_shape=jax.ShapeDtypeStruct((N, 1), jnp.float32),
    )(cnt2d)


def _k2_body(x_ref, w_ref, o_ref):
    xr = jnp.maximum(x_ref[...], 0.0)
    for r in range(R):
        o_ref[r] = jnp.dot(xr, w_ref[r], preferred_element_type=jnp.float32)


def _k2(prev, wrel):
    return pl.pallas_call(
        _k2_body,
        grid=(10,),
        in_specs=[
            pl.BlockSpec((1000, F), lambda i: (i, 0)),
            pl.BlockSpec((R, F, F), lambda i: (0, 0, 0)),
        ],
        out_specs=pl.BlockSpec((R, 1000, F), lambda i: (0, i, 0)),
        out_shape=jax.ShapeDtypeStruct((R, N, F), jnp.float32),
    )(prev, wrel)


def _k3_body(x_ref, a_ref, w_ref, b_ref, o_ref):
    xr = jnp.maximum(x_ref[...], 0.0)
    root = jnp.dot(xr, w_ref[...], preferred_element_type=jnp.float32)
    o_ref[...] = jnp.maximum(a_ref[0] + a_ref[1] + root + b_ref[...], 0.0)


def _k3(prev, agg2, wroot, b2):
    return pl.pallas_call(
        _k3_body,
        grid=(10,),
        in_specs=[
            pl.BlockSpec((1000, F), lambda i: (i, 0)),
            pl.BlockSpec((2, 1000, F), lambda i: (0, i, 0)),
            pl.BlockSpec((F, F), lambda i: (0, 0)),
            pl.BlockSpec((1, F), lambda i: (0, 0)),
        ],
        out_specs=pl.BlockSpec((1000, F), lambda i: (i, 0)),
        out_shape=jax.ShapeDtypeStruct((N, F), jnp.float32),
    )(prev, agg2, wroot, b2)


def _k4_body(x_ref, n_ref, d_ref, wl_ref, bl_ref, wr_ref, o_ref):
    x = x_ref[...]
    nb = n_ref[0] + n_ref[1]
    d = d_ref[...]
    acc = jnp.zeros(o_ref.shape, jnp.float32)
    for k in range(NDEG):
        hk = (jnp.dot(x, wl_ref[k], preferred_element_type=jnp.float32)
              + jnp.dot(nb, wr_ref[k], preferred_element_type=jnp.float32)
              + bl_ref[k][None, :])
        acc = acc + jnp.where(d == float(k), hk, 0.0)
    o_ref[...] = acc


def _k4(xm, nbr2, degc, wl, bl, wr):
    return pl.pallas_call(
        _k4_body,
        grid=(10,),
        in_specs=[
            pl.BlockSpec((1000, F), lambda i: (i, 0)),
            pl.BlockSpec((2, 1000, F), lambda i: (0, i, 0)),
            pl.BlockSpec((1000, 1), lambda i: (i, 0)),
            pl.BlockSpec((NDEG, F, F), lambda i: (0, 0, 0)),
            pl.BlockSpec((NDEG, F), lambda i: (0, 0)),
            pl.BlockSpec((NDEG, F, F), lambda i: (0, 0, 0)),
        ],
        out_specs=pl.BlockSpec((1000, F), lambda i: (i, 0)),
        out_shape=jax.ShapeDtypeStruct((N, F), jnp.float32),
    )(xm, nbr2, degc, wl, bl, wr)


def _k5_body(x_ref, b_ref, o_ref):
    @pl.when(pl.program_id(0) == 0)
    def _():
        o_ref[...] = jnp.zeros(o_ref.shape, jnp.float32)

    bi = b_ref[0]
    gids = lax.broadcasted_iota(jnp.int32, (NG, 1000), 0)
    oh = (jnp.broadcast_to(bi, (NG, 1000)) == gids).astype(jnp.float32)
    o_ref[...] += jnp.dot(oh, x_ref[...], preferred_element_type=jnp.float32)


def _k5(out2, batch3):
    return pl.pallas_call(
        _k5_body,
        grid=(10,),
        in_specs=[
            pl.BlockSpec((1000, F), lambda i: (i, 0)),
            pl.BlockSpec((1, 1, 1000), lambda i: (i, 0, 0)),
        ],
        out_specs=pl.BlockSpec((NG, F), lambda i: (0, 0)),
        out_shape=jax.ShapeDtypeStruct((NG, F), jnp.float32),
    )(out2, batch3)


def _k6_body(p_ref, w1_ref, b1_ref, w2_ref, b2_ref, o_ref):
    h = jnp.maximum(jnp.dot(p_ref[...], w1_ref[...],
                            preferred_element_type=jnp.float32) + b1_ref[...],
                    0.0)
    o_ref[...] = jnp.dot(h, w2_ref[...],
                         preferred_element_type=jnp.float32) + b2_ref[...]


def _k6(pooled, w1, b1_2, w2, b2_2):
    return pl.pallas_call(
        _k6_body,
        out_shape=jax.ShapeDtypeStruct((NG, w2.shape[1]), jnp.float32),
    )(pooled, w1, b1_2, w2, b2_2)


def kernel(x, edge_index, edge_attr, batch_idx, emb_W, emb_b, rgcn_Wrel,
           rgcn_Wroot, rgcn_b, mf_Wl, mf_bl, mf_Wr, head_W1, head_b1,
           head_W2, head_b2):
    src_f = edge_index[0]
    dst_f = edge_index[1]
    attr_t = edge_attr.T
    src3 = src_f.reshape(125, 1, 2560)
    dst3 = dst_f.reshape(125, 1, 2560)

    gidx3, key3 = _k0(attr_t, src3, dst3)
    gidx_f = gidx3.reshape(E)
    key_f = key3.reshape(E)

    z2 = jnp.zeros((640, F), jnp.float32)

    cnt_p = _s1(key_f)
    inv = _k1b(cnt_p.reshape(2, 1250, 128)).reshape(N * R)
    degc = _k1c(cnt_p.reshape(2, N, R))

    out = _k1(x, emb_W, emb_b.reshape(1, F))

    for _ in range(2):
        h_all = _k2(out, rgcn_Wrel).reshape(R * N, F)
        agg2 = _s3(gidx_f, key_f, dst_f, inv, h_all, z2).reshape(2, NPAD, F)
        xm = _k3(out, agg2, rgcn_Wroot, rgcn_b.reshape(1, F))
        nbr2 = _s4(src_f, dst_f, xm, z2).reshape(2, NPAD, F)
        out = _k4(xm, nbr2, degc, mf_Wl, mf_bl, mf_Wr)

    pooled = _k5(out, batch_idx.reshape(10, 1, 1000))
    return _k6(pooled, head_W1, head_b1.reshape(1, F),
               head_W2, head_b2.reshape(1, head_W2.shape[1]))


# speedup vs baseline: 5.7987x; 1.8044x over previous
import dataclasses
import functools

import jax
import jax.numpy as jnp
from jax import lax
from jax.experimental import pallas as pl
from jax.experimental.pallas import tpu as pltpu
from jax.experimental.pallas import tpu_sc as plsc

N = 10000
E = 320000
F = 128
R = 16
NG = 64
NDEG = 11

NC = 2
NS = 16
NW = NC * NS
EW = E // NW
CK = 80
NCH = EW // CK
NPAD = 10240

_MESH = plsc.VectorSubcoreMesh(core_axis_name="c", subcore_axis_name="s")

_SC_CP = pltpu.CompilerParams()
if "needs_layout_passes" in pltpu.CompilerParams.__dataclass_fields__:
    _SC_CP = dataclasses.replace(_SC_CP, needs_layout_passes=False)


def _k0_body(a_ref, s_ref, d_ref, g_ref, k_ref):
    best = a_ref[0:1, :]
    bidx = jnp.zeros(best.shape, jnp.int32)
    for k in range(1, R):
        v = a_ref[k:k + 1, :]
        gt = v > best
        bidx = jnp.where(gt, k, bidx)
        best = jnp.where(gt, v, best)
    src = s_ref[0]
    dst = d_ref[0]
    g_ref[0] = bidx * N + src
    k_ref[0] = dst * R + bidx


def _k0(attr_t, src3, dst3):
    w = 2560
    nblk = attr_t.shape[1] // w
    return pl.pallas_call(
        _k0_body,
        grid=(nblk,),
        in_specs=[
            pl.BlockSpec((R, w), lambda i: (0, i)),
            pl.BlockSpec((1, 1, w), lambda i: (i, 0, 0)),
            pl.BlockSpec((1, 1, w), lambda i: (i, 0, 0)),
        ],
        out_specs=[
            pl.BlockSpec((1, 1, w), lambda i: (i, 0, 0)),
            pl.BlockSpec((1, 1, w), lambda i: (i, 0, 0)),
        ],
        out_shape=[
            jax.ShapeDtypeStruct((nblk, 1, w), jnp.int32),
            jax.ShapeDtypeStruct((nblk, 1, w), jnp.int32),
        ],
    )(attr_t, src3, dst3)


@functools.partial(
    pl.kernel,
    mesh=_MESH,
    out_type=jax.ShapeDtypeStruct((NC * N * R,), jnp.float32),
    compiler_params=_SC_CP,
    scratch_types=[
        pltpu.VMEM((2000,), jnp.int32),
        pltpu.VMEM((10000,), jnp.float32),
    ],
)
def _s1(key_hbm, cnt_out, kbuf, cnt_v):
    c = lax.axis_index("c")
    s = lax.axis_index("s")
    lo = s * 10000
    ones16 = jnp.full((16,), 1.0, jnp.float32)
    zeros16 = jnp.zeros((16,), jnp.float32)

    @pl.loop(0, 625)
    def _(i):
        cnt_v[pl.ds(i * 16, 16)] = zeros16

    ehalf = E // NC

    @pl.loop(0, ehalf // 2000)
    def _(i):
        pltpu.sync_copy(key_hbm.at[pl.ds(c * ehalf + i * 2000, 2000)], kbuf)

        @pl.loop(0, 125)
        def _(m):
            kv16 = kbuf[pl.ds(m * 16, 16)]
            rel = kv16 - lo
            mask = (rel >= 0) & (rel < 10000)
            relc = jnp.where(mask, rel, 0)
            plsc.addupdate_scatter(cnt_v, [relc], ones16, mask=mask)

    pltpu.sync_copy(cnt_v, cnt_out.at[pl.ds(c * (N * R) + lo, 10000)])


NB = 4
NGRP = NCH // NB


@functools.partial(
    pl.kernel,
    mesh=_MESH,
    out_type=jax.ShapeDtypeStruct((NC * NPAD, F), jnp.float32),
    compiler_params=_SC_CP,
    scratch_types=(
        [pltpu.VMEM((NB, CK), jnp.int32)] * 3
        + [pltpu.VMEM((NB, CK), jnp.float32)]
        + [pltpu.VMEM((CK, F), jnp.float32)] * NB
        + [pltpu.SemaphoreType.DMA] * NB
        + [pltpu.VMEM_SHARED((NPAD, F), jnp.float32)]
    ),
)
def _s3(gidx_hbm, key_hbm, dst_hbm, inv_hbm, hall_hbm, z2_hbm, agg_out,
        gv2, kv2, dv2, nv2, r0, r1, r2, r3, m0, m1, m2, m3, agg_sp):
    c = lax.axis_index("c")
    s = lax.axis_index("s")
    w = s * NC + c
    rows_l = [r0, r1, r2, r3]
    sems = [m0, m1, m2, m3]

    pltpu.sync_copy(z2_hbm, agg_sp.at[pl.ds(s * 640, 640)])
    plsc.subcore_barrier()

    @pl.loop(0, NGRP)
    def _(g):
        base0 = w * EW + g * (NB * CK)
        hi = []
        for b in range(NB):
            sl = pl.ds(base0 + b * CK, CK)
            hi.append((
                pltpu.async_copy(gidx_hbm.at[sl], gv2.at[b], sems[b]),
                pltpu.async_copy(key_hbm.at[sl], kv2.at[b], sems[b]),
                pltpu.async_copy(dst_hbm.at[sl], dv2.at[b], sems[b]),
            ))
        hg = []
        for b in range(NB):
            for h in hi[b]:
                h.wait()
            hg.append((
                pltpu.async_copy(hall_hbm.at[gv2.at[b]], rows_l[b], sems[b]),
                pltpu.async_copy(inv_hbm.at[kv2.at[b]], nv2.at[b], sems[b]),
            ))
        hs = []
        for b in range(NB):
            for h in hg[b]:
                h.wait()
            rows = rows_l[b]
            nv = nv2.at[b]

            @pl.loop(0, CK)
            def _(j):
                j16 = jnp.full((16,), j, jnp.int32)
                bc = plsc.load_gather(nv, [j16])
                row = rows.at[j]
                for t in range(F // 16):
                    sl2 = pl.ds(t * 16, 16)
                    row[sl2] = row[sl2] * bc

            hs.append(pltpu.async_copy(rows, agg_sp.at[dv2.at[b]], sems[b],
                                       add=True))
        for b in range(NB):
            hs[b].wait()

    tb = w * EW + NGRP * (NB * CK)
    tsl = pl.ds(tb, CK)
    pltpu.sync_copy(gidx_hbm.at[tsl], gv2.at[0])
    pltpu.sync_copy(key_hbm.at[tsl], kv2.at[0])
    pltpu.sync_copy(dst_hbm.at[tsl], dv2.at[0])
    pltpu.sync_copy(inv_hbm.at[kv2.at[0]], nv2.at[0])
    pltpu.sync_copy(hall_hbm.at[gv2.at[0]], r0)
    nv_t = nv2.at[0]

    @pl.loop(0, CK)
    def _(j):
        j16 = jnp.full((16,), j, jnp.int32)
        bc = plsc.load_gather(nv_t, [j16])
        row = r0.at[j]
        for t in range(F // 16):
            sl2 = pl.ds(t * 16, 16)
            row[sl2] = row[sl2] * bc

    pltpu.sync_copy(r0, agg_sp.at[dv2.at[0]], add=True)

    plsc.subcore_barrier()
    pltpu.sync_copy(agg_sp.at[pl.ds(s * 640, 640)],
                    agg_out.at[pl.ds(c * NPAD + s * 640, 640)])


@functools.partial(
    pl.kernel,
    mesh=_MESH,
    out_type=jax.ShapeDtypeStruct((NC * NPAD, F), jnp.float32),
    scratch_types=(
        [pltpu.VMEM((NB, CK), jnp.int32)] * 2
        + [pltpu.VMEM((CK, F), jnp.float32)] * NB
        + [pltpu.SemaphoreType.DMA] * NB
        + [pltpu.VMEM_SHARED((NPAD, F), jnp.float32)]
    ),
)
def _s4(src_hbm, dst_hbm, xm_hbm, z2_hbm, nbr_out,
        sv2, dv2, r0, r1, r2, r3, m0, m1, m2, m3, nbr_sp):
    c = lax.axis_index("c")
    s = lax.axis_index("s")
    w = s * NC + c
    rows_l = [r0, r1, r2, r3]
    sems = [m0, m1, m2, m3]

    pltpu.sync_copy(z2_hbm, nbr_sp.at[pl.ds(s * 640, 640)])
    plsc.subcore_barrier()

    @pl.loop(0, NGRP)
    def _(g):
        base0 = w * EW + g * (NB * CK)
        hi = []
        for b in range(NB):
            sl = pl.ds(base0 + b * CK, CK)
            hi.append((
                pltpu.async_copy(src_hbm.at[sl], sv2.at[b], sems[b]),
                pltpu.async_copy(dst_hbm.at[sl], dv2.at[b], sems[b]),
            ))
        hg = []
        for b in range(NB):
            for h in hi[b]:
                h.wait()
            hg.append(pltpu.async_copy(xm_hbm.at[sv2.at[b]], rows_l[b],
                                       sems[b]))
        hs = []
        for b in range(NB):
            hg[b].wait()
            hs.append(pltpu.async_copy(rows_l[b], nbr_sp.at[dv2.at[b]],
                                       sems[b], add=True))
        for b in range(NB):
            hs[b].wait()

    tb = w * EW + NGRP * (NB * CK)
    tsl = pl.ds(tb, CK)
    pltpu.sync_copy(src_hbm.at[tsl], sv2.at[0])
    pltpu.sync_copy(dst_hbm.at[tsl], dv2.at[0])
    pltpu.sync_copy(xm_hbm.at[sv2.at[0]], r0)
    pltpu.sync_copy(r0, nbr_sp.at[dv2.at[0]], add=True)

    plsc.subcore_barrier()
    pltpu.sync_copy(nbr_sp.at[pl.ds(s * 640, 640)],
                    nbr_out.at[pl.ds(c * NPAD + s * 640, 640)])


def _k1_body(x_ref, w_ref, b_ref, o_ref):
    o_ref[...] = jnp.dot(x_ref[...], w_ref[...],
                         preferred_element_type=jnp.float32) + b_ref[...]


def _k1(x, w, b2):
    return pl.pallas_call(
        _k1_body,
        grid=(10,),
        in_specs=[
            pl.BlockSpec((1000, F), lambda i: (i, 0)),
            pl.BlockSpec((F, F), lambda i: (0, 0)),
            pl.BlockSpec((1, F), lambda i: (0, 0)),
        ],
        out_specs=pl.BlockSpec((1000, F), lambda i: (i, 0)),
        out_shape=jax.ShapeDtypeStruct((N, F), jnp.float32),
    )(x, w, b2)


def _k1b_body(c_ref, o_ref):
    cs = c_ref[0] + c_ref[1]
    o_ref[...] = 1.0 / jnp.maximum(cs, 1.0)


def _k1b(cnt2):
    return pl.pallas_call(
        _k1b_body,
        out_shape=jax.ShapeDtypeStruct((1250, 128), jnp.float32),
    )(cnt2)


def _k1c_body(c_ref, o_ref):
    d = jnp.sum(c_ref[0] + c_ref[1], axis=1, keepdims=True)
    o_ref[...] = jnp.clip(d, 0.0, float(NDEG - 1))


def _k1c(cnt2d):
    return pl.pallas_call(
        _k1c_body,
        grid=(10,),
        in_specs=[pl.BlockSpec((2, 1000, 16), lambda i: (0, i, 0))],
        out_specs=pl.BlockSpec((1000, 1), lambda i: (i, 0)),
        out_shape=jax.ShapeDtypeStruct((N, 1), jnp.float32),
    )(cnt2d)


def _k2_body(x_ref, w_ref, o_ref):
    xr = jnp.maximum(x_ref[...], 0.0)
    for r in range(R):
        o_ref[r] = jnp.dot(xr, w_ref[r], preferred_element_type=jnp.float32)


def _k2(prev, wrel):
    return pl.pallas_call(
        _k2_body,
        grid=(10,),
        in_specs=[
            pl.BlockSpec((1000, F), lambda i: (i, 0)),
            pl.BlockSpec((R, F, F), lambda i: (0, 0, 0)),
        ],
        out_specs=pl.BlockSpec((R, 1000, F), lambda i: (0, i, 0)),
        out_shape=jax.ShapeDtypeStruct((R, N, F), jnp.float32),
    )(prev, wrel)


def _k3_body(x_ref, a_ref, w_ref, b_ref, o_ref):
    xr = jnp.maximum(x_ref[...], 0.0)
    root = jnp.dot(xr, w_ref[...], preferred_element_type=jnp.float32)
    o_ref[...] = jnp.maximum(a_ref[0] + a_ref[1] + root + b_ref[...], 0.0)


def _k3(prev, agg2, wroot, b2):
    return pl.pallas_call(
        _k3_body,
        grid=(10,),
        in_specs=[
            pl.BlockSpec((1000, F), lambda i: (i, 0)),
            pl.BlockSpec((2, 1000, F), lambda i: (0, i, 0)),
            pl.BlockSpec((F, F), lambda i: (0, 0)),
            pl.BlockSpec((1, F), lambda i: (0, 0)),
        ],
        out_specs=pl.BlockSpec((1000, F), lambda i: (i, 0)),
        out_shape=jax.ShapeDtypeStruct((N, F), jnp.float32),
    )(prev, agg2, wroot, b2)


def _k4_body(x_ref, n_ref, d_ref, wl_ref, bl_ref, wr_ref, o_ref):
    x = x_ref[...]
    nb = n_ref[0] + n_ref[1]
    d = d_ref[...]
    acc = jnp.zeros(o_ref.shape, jnp.float32)
    for k in range(NDEG):
        hk = (jnp.dot(x, wl_ref[k], preferred_element_type=jnp.float32)
              + jnp.dot(nb, wr_ref[k], preferred_element_type=jnp.float32)
              + bl_ref[k][None, :])
        acc = acc + jnp.where(d == float(k), hk, 0.0)
    o_ref[...] = acc


def _k4(xm, nbr2, degc, wl, bl, wr):
    return pl.pallas_call(
        _k4_body,
        grid=(10,),
        in_specs=[
            pl.BlockSpec((1000, F), lambda i: (i, 0)),
            pl.BlockSpec((2, 1000, F), lambda i: (0, i, 0)),
            pl.BlockSpec((1000, 1), lambda i: (i, 0)),
            pl.BlockSpec((NDEG, F, F), lambda i: (0, 0, 0)),
            pl.BlockSpec((NDEG, F), lambda i: (0, 0)),
            pl.BlockSpec((NDEG, F, F), lambda i: (0, 0, 0)),
        ],
        out_specs=pl.BlockSpec((1000, F), lambda i: (i, 0)),
        out_shape=jax.ShapeDtypeStruct((N, F), jnp.float32),
    )(xm, nbr2, degc, wl, bl, wr)


def _k5_body(x_ref, b_ref, o_ref):
    @pl.when(pl.program_id(0) == 0)
    def _():
        o_ref[...] = jnp.zeros(o_ref.shape, jnp.float32)

    bi = b_ref[0]
    gids = lax.broadcasted_iota(jnp.int32, (NG, 1000), 0)
    oh = (jnp.broadcast_to(bi, (NG, 1000)) == gids).astype(jnp.float32)
    o_ref[...] += jnp.dot(oh, x_ref[...], preferred_element_type=jnp.float32)


def _k5(out2, batch3):
    return pl.pallas_call(
        _k5_body,
        grid=(10,),
        in_specs=[
            pl.BlockSpec((1000, F), lambda i: (i, 0)),
            pl.BlockSpec((1, 1, 1000), lambda i: (i, 0, 0)),
        ],
        out_specs=pl.BlockSpec((NG, F), lambda i: (0, 0)),
        out_shape=jax.ShapeDtypeStruct((NG, F), jnp.float32),
    )(out2, batch3)


def _k6_body(p_ref, w1_ref, b1_ref, w2_ref, b2_ref, o_ref):
    h = jnp.maximum(jnp.dot(p_ref[...], w1_ref[...],
                            preferred_element_type=jnp.float32) + b1_ref[...],
                    0.0)
    o_ref[...] = jnp.dot(h, w2_ref[...],
                         preferred_element_type=jnp.float32) + b2_ref[...]


def _k6(pooled, w1, b1_2, w2, b2_2):
    return pl.pallas_call(
        _k6_body,
        out_shape=jax.ShapeDtypeStruct((NG, w2.shape[1]), jnp.float32),
    )(pooled, w1, b1_2, w2, b2_2)


def kernel(x, edge_index, edge_attr, batch_idx, emb_W, emb_b, rgcn_Wrel,
           rgcn_Wroot, rgcn_b, mf_Wl, mf_bl, mf_Wr, head_W1, head_b1,
           head_W2, head_b2):
    src_f = edge_index[0]
    dst_f = edge_index[1]
    attr_t = edge_attr.T
    src3 = src_f.reshape(125, 1, 2560)
    dst3 = dst_f.reshape(125, 1, 2560)

    gidx3, key3 = _k0(attr_t, src3, dst3)
    gidx_f = gidx3.reshape(E)
    key_f = key3.reshape(E)

    z2 = jnp.zeros((640, F), jnp.float32)

    cnt_p = _s1(key_f)
    inv = _k1b(cnt_p.reshape(2, 1250, 128)).reshape(N * R)
    degc = _k1c(cnt_p.reshape(2, N, R))

    out = _k1(x, emb_W, emb_b.reshape(1, F))

    for _ in range(2):
        h_all = _k2(out, rgcn_Wrel).reshape(R * N, F)
        agg2 = _s3(gidx_f, key_f, dst_f, inv, h_all, z2).reshape(2, NPAD, F)
        xm = _k3(out, agg2, rgcn_Wroot, rgcn_b.reshape(1, F))
        nbr2 = _s4(src_f, dst_f, xm, z2).reshape(2, NPAD, F)
        out = _k4(xm, nbr2, degc, mf_Wl, mf_bl, mf_Wr)

    pooled = _k5(out, batch_idx.reshape(10, 1, 1000))
    return _k6(pooled, head_W1, head_b1.reshape(1, F),
               head_W2, head_b2.reshape(1, head_W2.shape[1]))
